# baseline (speedup 1.0000x reference)
import functools

import jax
import jax.numpy as jnp
from jax import lax
from jax.experimental import pallas as pl
from jax.experimental.pallas import tpu as pltpu
from jax.experimental.pallas import tpu_sc as plsc

_L = 16
_NSUB = 16
_C = 80
_RPT = 624
_RLAST = 640
_ACCR = 641
_PW = 664
_DUMP = 640

_GATHER_DNUMS = lax.GatherDimensionNumbers(
    offset_dims=(), collapsed_slice_dims=(0,), start_index_map=(0,))


def _mm_nt(x, w):
    return lax.dot_general(x, w, (((1,), (1,)), ((), ())),
                           preferred_element_type=jnp.float32)


def _pre_body(t_ref, a_ref, wt_ref, wat_ref, wa_ref, tp_ref, ap_ref, ao_ref):
    tp_ref[...] = _mm_nt(t_ref[...], wt_ref[...])
    ap_ref[...] = _mm_nt(a_ref[...], wat_ref[...])
    ao_ref[...] = jnp.dot(a_ref[...], wa_ref[...],
                          preferred_element_type=jnp.float32)


def _post_body(t_ref, v_ref, s1_ref, s2_ref, wav_ref, w1_ref, w2_ref,
               tu_ref, vu_ref):
    d = t_ref.shape[1]
    out1 = _mm_nt(s1_ref[...], wav_ref[...])
    tu_ref[...] = _mm_nt(t_ref[...], w1_ref[:, :d]) + _mm_nt(out1, w1_ref[:, d:])
    vu_ref[...] = _mm_nt(v_ref[...], w2_ref[:, :d]) + _mm_nt(s2_ref[...], w2_ref[:, d:])


def _seg_body(ta_t, tb_t, ptr_t, la_t, lb_t, ta_v, tb_v, ptr_v, la_v, lb_v,
              s1, s2, ptrw,
              idxa0, idxb0, rowsa0, rowsb0, idxa1, idxb1, rowsa1, rowsb1,
              acc, si0, si1, sg0, sg1):
    core = lax.axis_index("core")
    sub = lax.axis_index("sub")
    iota = lax.iota(jnp.int32, _L)
    zero16 = jnp.zeros((_L,), jnp.float32)
    czero = jnp.zeros((_L,), jnp.int32)
    bufs = ((idxa0, idxb0, rowsa0, rowsb0, si0, sg0),
            (idxa1, idxb1, rowsa1, rowsb1, si1, sg1))

    n0 = sub * _RPT
    rpt = jnp.where(sub < _NSUB - 1, _RPT, _RLAST)

    @pl.loop(0, _ACCR)
    def _zero(r):
        for d8 in range(8):
            acc[r, pl.ds(d8 * _L, _L)] = zero16

    def run_side(taba, tabb, ptr, la, lb, out):
        pltpu.sync_copy(ptr.at[pl.ds(pl.multiple_of(n0, 8), _PW)], ptrw)
        e_start = jnp.min(plsc.load_gather(ptrw, [iota]))
        e_end = jnp.min(plsc.load_gather(ptrw, [rpt + iota]))
        e0 = jnp.bitwise_and(e_start, -8)
        nch = (e_end - e0 + _C - 1) // _C

        def cbase(c):
            return pl.multiple_of(e0 + c * _C, 8)

        def issue_idx(c, b):
            ia, ib, _, _, si, _ = bufs[b]
            pltpu.async_copy(la.at[pl.ds(cbase(c), _C)], ia, si)
            pltpu.async_copy(lb.at[pl.ds(cbase(c), _C)], ib, si)

        def issue_gather(c, b):
            ia, ib, ra, rb, si, sg = bufs[b]
            pltpu.make_async_copy(la.at[pl.ds(cbase(c), _C)], ia, si).wait()
            pltpu.make_async_copy(lb.at[pl.ds(cbase(c), _C)], ib, si).wait()
            pltpu.async_copy(taba.at[ia], ra, sg)
            pltpu.async_copy(tabb.at[ib], rb, sg)

        def slot(c, b):
            @pl.when(c < nch)
            def _():
                ia, ib, ra, rb, si, sg = bufs[b]
                pltpu.make_async_copy(taba.at[ia], ra, sg).wait()
                pltpu.make_async_copy(tabb.at[ib], rb, sg).wait()

                @pl.when(c + 2 < nch)
                def _():
                    issue_idx(c + 2, b)

                @pl.when(c + 1 < nch)
                def _():
                    issue_gather(c + 1, b ^ 1)

                base = cbase(c)

                @pl.loop(0, _C // _L)
                def _group(k):
                    e_vec = base + k * _L + iota
                    lo = czero
                    hi = czero + _PW
                    for _ in range(10):
                        mid = jnp.right_shift(lo + hi, 1)
                        val = plsc.load_gather(ptrw, [mid])
                        cnd = val <= e_vec
                        lo = jnp.where(cnd, mid + 1, lo)
                        hi = jnp.where(cnd, hi, mid)
                    segl = lo - 1
                    valid = ((e_vec >= e_start) & (e_vec < e_end)
                             & (segl >= 0) & (segl < rpt))
                    segv = jnp.where(valid, segl, _DUMP)
                    for j in range(_L):
                        jj = k * _L + j
                        row = lax.gather(
                            segv, (czero + j)[:, None], _GATHER_DNUMS, (1,),
                            mode=lax.GatherScatterMode.PROMISE_IN_BOUNDS)
                        for d8 in range(8):
                            sl = pl.ds(d8 * _L, _L)
                            val = ra[jj, sl] * rb[jj, sl]
                            plsc.addupdate_scatter(acc, [row, iota + d8 * _L],
                                                   val)

        @pl.when(nch > 0)
        def _():
            issue_idx(0, 0)

            @pl.when(nch > 1)
            def _():
                issue_idx(1, 1)

            issue_gather(0, 0)

            @pl.loop(0, (nch + 1) // 2)
            def _pair(t):
                slot(2 * t, 0)
                slot(2 * t + 1, 1)

        @pl.when(sub < _NSUB - 1)
        def _():
            pltpu.sync_copy(acc.at[pl.ds(0, _RPT)],
                            out.at[pl.ds(pl.multiple_of(n0, 8), _RPT)])

        @pl.when(sub == _NSUB - 1)
        def _():
            pltpu.sync_copy(acc.at[pl.ds(0, _RLAST)],
                            out.at[pl.ds(pl.multiple_of(n0, 8), _RLAST)])

    @pl.when(core == 0)
    def _():
        run_side(ta_t, tb_t, ptr_t, la_t, lb_t, s1)

    @pl.when(core == 1)
    def _():
        run_side(ta_v, tb_v, ptr_v, la_v, lb_v, s2)


def kernel(t_embed, v_embed, a_embed, wt, wa_t, w1, w2, wa, wa_v,
           ptr_t, a_list_t, v_list_t, ptr_v, a_list_v, t_list_v):
    n, d = t_embed.shape
    e = a_list_t.shape[0]
    f32 = jnp.float32

    bn = 2000
    grid = (n // bn,)
    row_spec = pl.BlockSpec((bn, d), lambda i: (i, 0))
    w_spec = pl.BlockSpec((d, d), lambda i: (0, 0))
    tp, ap, a_out = pl.pallas_call(
        _pre_body,
        grid=grid,
        in_specs=[row_spec, row_spec, w_spec, w_spec, w_spec],
        out_specs=[row_spec, row_spec, row_spec],
        out_shape=[jax.ShapeDtypeStruct((n, d), f32)] * 3,
    )(t_embed, a_embed, wt, wa_t, wa)

    ptr_pad = (_NSUB - 1) * _RPT + _PW
    e_pad = e + 2 * _C

    def pad_ptr(p):
        return jnp.concatenate(
            [p.astype(jnp.int32), jnp.full((ptr_pad - p.shape[0],), e, jnp.int32)])

    def pad_list(x):
        return jnp.concatenate(
            [x.astype(jnp.int32), jnp.zeros((e_pad - x.shape[0],), jnp.int32)])

    seg_kernel = _make_seg_kernel(n, d)
    s1, s2 = seg_kernel(
        a_embed, v_embed, pad_ptr(ptr_t), pad_list(a_list_t), pad_list(v_list_t),
        ap, tp, pad_ptr(ptr_v), pad_list(a_list_v), pad_list(t_list_v))

    w2d_spec = pl.BlockSpec((d, 2 * d), lambda i: (0, 0))
    t_up, v_up = pl.pallas_call(
        _post_body,
        grid=grid,
        in_specs=[row_spec, row_spec, row_spec, row_spec, w_spec,
                  w2d_spec, w2d_spec],
        out_specs=[row_spec, row_spec],
        out_shape=[jax.ShapeDtypeStruct((n, d), f32)] * 2,
    )(t_embed, v_embed, s1, s2, wa_v, w1, w2)

    return (t_up, v_up, a_out)


def _make_seg_kernel(n, d):
    f32 = jnp.float32
    return pl.kernel(
        _seg_body,
        out_type=(jax.ShapeDtypeStruct((n, d), f32),
                  jax.ShapeDtypeStruct((n, d), f32)),
        mesh=plsc.VectorSubcoreMesh(core_axis_name="core", subcore_axis_name="sub",
                                    num_cores=2, num_subcores=_NSUB),
        compiler_params=pltpu.CompilerParams(needs_layout_passes=False,
                                             disable_bounds_checks=True),
        scratch_types=[
            pltpu.VMEM((_PW,), jnp.int32),
            pltpu.VMEM((_C,), jnp.int32),
            pltpu.VMEM((_C,), jnp.int32),
            pltpu.VMEM((_C, d), f32),
            pltpu.VMEM((_C, d), f32),
            pltpu.VMEM((_C,), jnp.int32),
            pltpu.VMEM((_C,), jnp.int32),
            pltpu.VMEM((_C, d), f32),
            pltpu.VMEM((_C, d), f32),
            pltpu.VMEM((_ACCR, d), f32),
            pltpu.SemaphoreType.DMA,
            pltpu.SemaphoreType.DMA,
            pltpu.SemaphoreType.DMA,
            pltpu.SemaphoreType.DMA,
        ],
    )


# speedup vs baseline: 1.0058x; 1.0058x over previous
import functools

import jax
import jax.numpy as jnp
from jax import lax
from jax.experimental import pallas as pl
from jax.experimental.pallas import tpu as pltpu
from jax.experimental.pallas import tpu_sc as plsc

_L = 16
_NSUB = 16
_C = 64
_CI = 2 * _C
_RPT = 624
_RLAST = 640
_ACCR = 641
_PW = 664
_DUMP = 640

_GATHER_DNUMS = lax.GatherDimensionNumbers(
    offset_dims=(), collapsed_slice_dims=(0,), start_index_map=(0,))


def _mm_nt(x, w):
    return lax.dot_general(x, w, (((1,), (1,)), ((), ())),
                           preferred_element_type=jnp.float32)


def _pre_body(t_ref, a_ref, wt_ref, wat_ref, wa_ref, tp_ref, ap_ref, ao_ref):
    tp_ref[...] = _mm_nt(t_ref[...], wt_ref[...])
    ap_ref[...] = _mm_nt(a_ref[...], wat_ref[...])
    ao_ref[...] = jnp.dot(a_ref[...], wa_ref[...],
                          preferred_element_type=jnp.float32)


def _post_body(t_ref, v_ref, s1_ref, s2_ref, wav_ref, w1_ref, w2_ref,
               tu_ref, vu_ref):
    d = t_ref.shape[1]
    out1 = _mm_nt(s1_ref[...], wav_ref[...])
    tu_ref[...] = _mm_nt(t_ref[...], w1_ref[:, :d]) + _mm_nt(out1, w1_ref[:, d:])
    vu_ref[...] = _mm_nt(v_ref[...], w2_ref[:, :d]) + _mm_nt(s2_ref[...], w2_ref[:, d:])


def _seg_body(tab_t, ptr_t, xl_t, tab_v, ptr_v, xl_v,
              s1, s2, ptrw, idx0, rows0, idx1, rows1,
              acc, si0, si1, sg0, sg1):
    core = lax.axis_index("core")
    sub = lax.axis_index("sub")
    iota = lax.iota(jnp.int32, _L)
    zero16 = jnp.zeros((_L,), jnp.float32)
    czero = jnp.zeros((_L,), jnp.int32)
    bufs = ((idx0, rows0, si0, sg0), (idx1, rows1, si1, sg1))

    n0 = sub * _RPT
    rpt = jnp.where(sub < _NSUB - 1, _RPT, _RLAST)

    @pl.loop(0, _ACCR)
    def _zero(r):
        for d8 in range(8):
            acc[r, pl.ds(d8 * _L, _L)] = zero16

    def run_side(tab, ptr, xl, out):
        pltpu.sync_copy(ptr.at[pl.ds(pl.multiple_of(n0, 8), _PW)], ptrw)
        e_start = jnp.min(plsc.load_gather(ptrw, [iota]))
        e_end = jnp.min(plsc.load_gather(ptrw, [rpt + iota]))
        e0 = jnp.bitwise_and(e_start, -_C)
        nch = (e_end - e0 + _C - 1) // _C
        cb0 = jnp.right_shift(e0, 6) * _CI

        def issue_idx(c, b):
            ia, _, si, _ = bufs[b]
            pltpu.async_copy(xl.at[pl.ds(pl.multiple_of(cb0 + c * _CI, 8), _CI)],
                             ia, si)

        def issue_gather(c, b):
            ia, ra, si, sg = bufs[b]
            pltpu.make_async_copy(
                xl.at[pl.ds(pl.multiple_of(cb0 + c * _CI, 8), _CI)], ia,
                si).wait()
            pltpu.async_copy(tab.at[ia], ra, sg)

        def slot(c, b):
            @pl.when(c < nch)
            def _():
                ia, ra, si, sg = bufs[b]
                pltpu.make_async_copy(tab.at[ia], ra, sg).wait()

                @pl.when(c + 2 < nch)
                def _():
                    issue_idx(c + 2, b)

                @pl.when(c + 1 < nch)
                def _():
                    issue_gather(c + 1, b ^ 1)

                base = e0 + c * _C

                @pl.loop(0, _C // _L)
                def _group(k):
                    e_vec = base + k * _L + iota
                    lo = czero
                    hi = czero + _PW
                    for _ in range(10):
                        mid = jnp.right_shift(lo + hi, 1)
                        val = plsc.load_gather(ptrw, [mid])
                        cnd = val <= e_vec
                        lo = jnp.where(cnd, mid + 1, lo)
                        hi = jnp.where(cnd, hi, mid)
                    segl = lo - 1
                    valid = ((e_vec >= e_start) & (e_vec < e_end)
                             & (segl >= 0) & (segl < rpt))
                    segv = jnp.where(valid, segl, _DUMP)
                    for j in range(_L):
                        jj = k * _L + j
                        row = lax.gather(
                            segv, (czero + j)[:, None], _GATHER_DNUMS, (1,),
                            mode=lax.GatherScatterMode.PROMISE_IN_BOUNDS)
                        for d8 in range(8):
                            sl = pl.ds(d8 * _L, _L)
                            val = ra[jj, sl] * ra[_C + jj, sl]
                            plsc.addupdate_scatter(acc, [row, iota + d8 * _L],
                                                   val)

        @pl.when(nch > 0)
        def _():
            issue_idx(0, 0)

            @pl.when(nch > 1)
            def _():
                issue_idx(1, 1)

            issue_gather(0, 0)

            @pl.loop(0, (nch + 1) // 2)
            def _pair(t):
                slot(2 * t, 0)
                slot(2 * t + 1, 1)

        @pl.when(sub < _NSUB - 1)
        def _():
            pltpu.sync_copy(acc.at[pl.ds(0, _RPT)],
                            out.at[pl.ds(pl.multiple_of(n0, 8), _RPT)])

        @pl.when(sub == _NSUB - 1)
        def _():
            pltpu.sync_copy(acc.at[pl.ds(0, _RLAST)],
                            out.at[pl.ds(pl.multiple_of(n0, 8), _RLAST)])

    @pl.when(core == 0)
    def _():
        run_side(tab_t, ptr_t, xl_t, s1)

    @pl.when(core == 1)
    def _():
        run_side(tab_v, ptr_v, xl_v, s2)


def _make_seg_kernel(n, d):
    f32 = jnp.float32
    return pl.kernel(
        _seg_body,
        out_type=(jax.ShapeDtypeStruct((n, d), f32),
                  jax.ShapeDtypeStruct((n, d), f32)),
        mesh=plsc.VectorSubcoreMesh(core_axis_name="core", subcore_axis_name="sub",
                                    num_cores=2, num_subcores=_NSUB),
        compiler_params=pltpu.CompilerParams(needs_layout_passes=False,
                                             disable_bounds_checks=True),
        scratch_types=[
            pltpu.VMEM((_PW,), jnp.int32),
            pltpu.VMEM((_CI,), jnp.int32),
            pltpu.VMEM((_CI, d), f32),
            pltpu.VMEM((_CI,), jnp.int32),
            pltpu.VMEM((_CI, d), f32),
            pltpu.VMEM((_ACCR, d), f32),
            pltpu.SemaphoreType.DMA,
            pltpu.SemaphoreType.DMA,
            pltpu.SemaphoreType.DMA,
            pltpu.SemaphoreType.DMA,
        ],
    )


def kernel(t_embed, v_embed, a_embed, wt, wa_t, w1, w2, wa, wa_v,
           ptr_t, a_list_t, v_list_t, ptr_v, a_list_v, t_list_v):
    n, d = t_embed.shape
    e = a_list_t.shape[0]
    f32 = jnp.float32

    bn = 2000
    grid = (n // bn,)
    row_spec = pl.BlockSpec((bn, d), lambda i: (i, 0))
    w_spec = pl.BlockSpec((d, d), lambda i: (0, 0))
    tp, ap, a_out = pl.pallas_call(
        _pre_body,
        grid=grid,
        in_specs=[row_spec, row_spec, w_spec, w_spec, w_spec],
        out_specs=[row_spec, row_spec, row_spec],
        out_shape=[jax.ShapeDtypeStruct((n, d), f32)] * 3,
    )(t_embed, a_embed, wt, wa_t, wa)

    ptr_pad = (_NSUB - 1) * _RPT + _PW

    def pad_ptr(p):
        return jnp.concatenate(
            [p.astype(jnp.int32), jnp.full((ptr_pad - p.shape[0],), e, jnp.int32)])

    def pack_idx(lsta, lstb):
        za = jnp.concatenate([lsta.astype(jnp.int32), jnp.zeros((_C,), jnp.int32)])
        zb = jnp.concatenate([lstb.astype(jnp.int32), jnp.zeros((_C,), jnp.int32)])
        return jnp.concatenate(
            [za.reshape(-1, _C), zb.reshape(-1, _C) + n], axis=1).reshape(-1)

    seg_kernel = _make_seg_kernel(n, d)
    s1, s2 = seg_kernel(
        jnp.concatenate([a_embed, v_embed], axis=0), pad_ptr(ptr_t),
        pack_idx(a_list_t, v_list_t),
        jnp.concatenate([ap, tp], axis=0), pad_ptr(ptr_v),
        pack_idx(a_list_v, t_list_v))

    w2d_spec = pl.BlockSpec((d, 2 * d), lambda i: (0, 0))
    t_up, v_up = pl.pallas_call(
        _post_body,
        grid=grid,
        in_specs=[row_spec, row_spec, row_spec, row_spec, w_spec,
                  w2d_spec, w2d_spec],
        out_specs=[row_spec, row_spec],
        out_shape=[jax.ShapeDtypeStruct((n, d), f32)] * 2,
    )(t_embed, v_embed, s1, s2, wa_v, w1, w2)

    return (t_up, v_up, a_out)
